# baseline (speedup 1.0000x reference)
import jax
import jax.numpy as jnp
from jax import lax
from jax.experimental import pallas as pl
from jax.experimental.pallas import tpu as pltpu
from jax.experimental.pallas import tpu_sc as plsc

_DIM = 32
_LANES = 16
_NUM_CORES = 2
_NUM_SUBCORES = 16
_NUM_WORKERS = _NUM_CORES * _NUM_SUBCORES
_CHUNK = 128
_VPC = _CHUNK // _LANES
_BLK = _DIM * _CHUNK


def _transe_body(hidx_hbm, ridx_hbm, tidx_hbm, entf_hbm, relf_hbm, dummy_hbm,
                 out_hbm,
                 hidx_v, ridx_v, tidx_v,
                 hix0, rix0, tix0, hix1, rix1, tix1,
                 hbuf0, rbuf0, tbuf0, hbuf1, rbuf1, tbuf1,
                 out_v, sem0, sem1):
    wid = lax.axis_index("s") * _NUM_CORES + lax.axis_index("c")
    nchunks = hidx_v.shape[0]
    n = nchunks * _CHUNK
    nrows = entf_hbm.shape[0] // _DIM

    pltpu.sync_copy(hidx_hbm.at[wid], hidx_v)
    pltpu.sync_copy(ridx_hbm.at[wid], ridx_v)
    pltpu.sync_copy(tidx_hbm.at[wid], tidx_v)

    bufs = (
        (hix0, rix0, tix0, hbuf0, rbuf0, tbuf0, sem0),
        (hix1, rix1, tix1, hbuf1, rbuf1, tbuf1, sem1),
    )

    def build_and_fire(c, slot):
        hix, rix, tix, hbuf, rbuf, tbuf, sem = bufs[slot]
        vecs = []
        for v in range(_VPC):
            ds_v = pl.ds(v * _LANES, _LANES)
            vecs.append((hidx_v[c, ds_v], ridx_v[c, ds_v], tidx_v[c, ds_v]))

        def d_body(d, carry):
            off = d * nrows
            base = d * _CHUNK
            for v in range(_VPC):
                ds_v = pl.ds(base + v * _LANES, _LANES)
                hb, rb, tb = vecs[v]
                hix[ds_v] = hb + off
                rix[ds_v] = rb + off
                tix[ds_v] = tb + off
            return carry

        lax.fori_loop(0, _DIM, d_body, 0)
        pltpu.async_copy(entf_hbm.at[hix], hbuf, sem)
        pltpu.async_copy(relf_hbm.at[rix], rbuf, sem)
        pltpu.async_copy(entf_hbm.at[tix], tbuf, sem)

    def drain(slot):
        _, _, _, hbuf, rbuf, tbuf, sem = bufs[slot]
        pltpu.make_async_copy(dummy_hbm, hbuf, sem).wait()
        pltpu.make_async_copy(dummy_hbm, rbuf, sem).wait()
        pltpu.make_async_copy(dummy_hbm, tbuf, sem).wait()

    def reduce(c, slot):
        _, _, _, hbuf, rbuf, tbuf, _ = bufs[slot]

        def d_body(d, accs):
            base = d * _CHUNK
            out = []
            for v in range(_VPC):
                ds_v = pl.ds(base + v * _LANES, _LANES)
                a = jnp.abs(hbuf[ds_v] + rbuf[ds_v] - tbuf[ds_v])
                out.append(accs[v] + a)
            return tuple(out)

        zero = jnp.zeros((_LANES,), jnp.float32)
        accs = lax.fori_loop(0, _DIM, d_body, (zero,) * _VPC)
        for v in range(_VPC):
            out_v[pl.ds(c * _CHUNK + v * _LANES, _LANES)] = accs[v]

    build_and_fire(0, 0)
    build_and_fire(1, 1)

    def pair_body(p, carry):
        c0 = 2 * p
        drain(0)
        reduce(c0, 0)

        @pl.when(c0 + 2 < nchunks)
        def _():
            build_and_fire(c0 + 2, 0)

        drain(1)
        reduce(c0 + 1, 1)

        @pl.when(c0 + 3 < nchunks)
        def _():
            build_and_fire(c0 + 3, 1)

        return carry

    lax.fori_loop(0, nchunks // 2, pair_body, 0)

    pltpu.sync_copy(out_v, out_hbm.at[pl.ds(wid * n, n)])


def kernel(pos_triples, neg_triples, entity_emb, relation_emb):
    trip = jnp.concatenate([pos_triples, neg_triples], axis=0)
    total = trip.shape[0]
    n = total // _NUM_WORKERS
    nchunks = n // _CHUNK
    hidx = trip[:, 0].reshape(_NUM_WORKERS, nchunks, _CHUNK)
    ridx = trip[:, 1].reshape(_NUM_WORKERS, nchunks, _CHUNK)
    tidx = trip[:, 2].reshape(_NUM_WORKERS, nchunks, _CHUNK)
    entf = entity_emb.T.reshape(-1)
    relf = relation_emb.T.reshape(-1)
    dummy = jnp.zeros((_BLK,), jnp.float32)

    mesh = plsc.VectorSubcoreMesh(core_axis_name="c", subcore_axis_name="s")
    ix = pltpu.VMEM((_BLK,), jnp.int32)
    buf = pltpu.VMEM((_BLK,), jnp.float32)
    f = pl.kernel(
        _transe_body,
        mesh=mesh,
        compiler_params=pltpu.CompilerParams(use_tc_tiling_on_sc=False),
        out_type=jax.ShapeDtypeStruct((total,), jnp.float32),
        scratch_types=[
            pltpu.VMEM((nchunks, _CHUNK), jnp.int32),
            pltpu.VMEM((nchunks, _CHUNK), jnp.int32),
            pltpu.VMEM((nchunks, _CHUNK), jnp.int32),
            ix, ix, ix, ix, ix, ix,
            buf, buf, buf, buf, buf, buf,
            pltpu.VMEM((n,), jnp.float32),
            pltpu.SemaphoreType.DMA,
            pltpu.SemaphoreType.DMA,
        ],
    )
    return f(hidx, ridx, tidx, entf, relf, dummy)


# speedup vs baseline: 1.4419x; 1.4419x over previous
import jax
import jax.numpy as jnp
from jax import lax
from jax.experimental import pallas as pl
from jax.experimental.pallas import tpu as pltpu
from jax.experimental.pallas import tpu_sc as plsc

_DIM = 32
_LANES = 16
_NUM_CORES = 2
_NUM_SUBCORES = 16
_NUM_WORKERS = _NUM_CORES * _NUM_SUBCORES
_CHUNK = 128
_VPC = _CHUNK // _LANES
_BLK = _DIM * _CHUNK


def _transe_body(hidx_hbm, ridx_hbm, tidx_hbm, entf_hbm, relf_hbm, dummy_hbm,
                 out_hbm,
                 hidx_v, ridx_v, tidx_v,
                 hix0, rix0, tix0, hix1, rix1, tix1,
                 hbuf0, rbuf0, tbuf0, hbuf1, rbuf1, tbuf1,
                 out_v, sem0, sem1):
    wid = lax.axis_index("s") * _NUM_CORES + lax.axis_index("c")
    nchunks = hidx_v.shape[0]
    n = nchunks * _CHUNK
    nrows = entf_hbm.shape[0] // _DIM

    pltpu.sync_copy(hidx_hbm.at[wid], hidx_v)
    pltpu.sync_copy(ridx_hbm.at[wid], ridx_v)
    pltpu.sync_copy(tidx_hbm.at[wid], tidx_v)

    bufs = (
        (hix0, rix0, tix0, hbuf0, rbuf0, tbuf0, sem0),
        (hix1, rix1, tix1, hbuf1, rbuf1, tbuf1, sem1),
    )

    def build_and_fire(c, slot):
        hix, rix, tix, hbuf, rbuf, tbuf, sem = bufs[slot]
        vecs = []
        for v in range(_VPC):
            ds_v = pl.ds(v * _LANES, _LANES)
            vecs.append((hidx_v[c, ds_v], ridx_v[c, ds_v], tidx_v[c, ds_v]))

        def d_body(d, carry):
            off = d * nrows
            base = d * _CHUNK
            for v in range(_VPC):
                ds_v = pl.ds(base + v * _LANES, _LANES)
                hb, rb, tb = vecs[v]
                hix[ds_v] = hb + off
                rix[ds_v] = rb + off
                tix[ds_v] = tb + off
            return carry

        lax.fori_loop(0, _DIM, d_body, 0)
        pltpu.async_copy(entf_hbm.at[hix], hbuf, sem)
        pltpu.async_copy(relf_hbm.at[rix], rbuf, sem)
        pltpu.async_copy(entf_hbm.at[tix], tbuf, sem)

    def drain(slot):
        _, _, _, hbuf, rbuf, tbuf, sem = bufs[slot]
        pltpu.make_async_copy(dummy_hbm, hbuf, sem).wait()
        pltpu.make_async_copy(dummy_hbm, rbuf, sem).wait()
        pltpu.make_async_copy(dummy_hbm, tbuf, sem).wait()

    def reduce(c, slot):
        _, _, _, hbuf, rbuf, tbuf, _ = bufs[slot]

        def d_body(d, accs):
            base = d * _CHUNK
            out = []
            for v in range(_VPC):
                ds_v = pl.ds(base + v * _LANES, _LANES)
                a = jnp.abs(hbuf[ds_v] + rbuf[ds_v] - tbuf[ds_v])
                out.append(accs[v] + a)
            return tuple(out)

        zero = jnp.zeros((_LANES,), jnp.float32)
        accs = lax.fori_loop(0, _DIM, d_body, (zero,) * _VPC)
        for v in range(_VPC):
            out_v[pl.ds(c * _CHUNK + v * _LANES, _LANES)] = accs[v]

    build_and_fire(0, 0)
    build_and_fire(1, 1)

    def pair_body(p, carry):
        c0 = 2 * p
        drain(0)
        reduce(c0, 0)

        @pl.when(c0 + 2 < nchunks)
        def _():
            build_and_fire(c0 + 2, 0)

        drain(1)
        reduce(c0 + 1, 1)

        @pl.when(c0 + 3 < nchunks)
        def _():
            build_and_fire(c0 + 3, 1)

        return carry

    lax.fori_loop(0, nchunks // 2, pair_body, 0)

    pltpu.sync_copy(out_v, out_hbm.at[pl.ds(wid * n, n)])


def _dmajor_flat(table):
    return jnp.concatenate([table[:, d] for d in range(table.shape[1])])


def kernel(pos_triples, neg_triples, entity_emb, relation_emb):
    trip = jnp.concatenate([pos_triples, neg_triples], axis=0)
    total = trip.shape[0]
    n = total // _NUM_WORKERS
    nchunks = n // _CHUNK
    hidx = trip[:, 0].reshape(_NUM_WORKERS, nchunks, _CHUNK)
    ridx = trip[:, 1].reshape(_NUM_WORKERS, nchunks, _CHUNK)
    tidx = trip[:, 2].reshape(_NUM_WORKERS, nchunks, _CHUNK)
    entf = _dmajor_flat(entity_emb)
    relf = _dmajor_flat(relation_emb)
    dummy = jnp.zeros((_BLK,), jnp.float32)

    mesh = plsc.VectorSubcoreMesh(core_axis_name="c", subcore_axis_name="s")
    ix = pltpu.VMEM((_BLK,), jnp.int32)
    buf = pltpu.VMEM((_BLK,), jnp.float32)
    f = pl.kernel(
        _transe_body,
        mesh=mesh,
        compiler_params=pltpu.CompilerParams(use_tc_tiling_on_sc=False),
        out_type=jax.ShapeDtypeStruct((total,), jnp.float32),
        scratch_types=[
            pltpu.VMEM((nchunks, _CHUNK), jnp.int32),
            pltpu.VMEM((nchunks, _CHUNK), jnp.int32),
            pltpu.VMEM((nchunks, _CHUNK), jnp.int32),
            ix, ix, ix, ix, ix, ix,
            buf, buf, buf, buf, buf, buf,
            pltpu.VMEM((n,), jnp.float32),
            pltpu.SemaphoreType.DMA,
            pltpu.SemaphoreType.DMA,
        ],
    )
    return f(hidx, ridx, tidx, entf, relf, dummy)
